# baseline (speedup 1.0000x reference)
import functools

import jax
import jax.numpy as jnp
from jax import lax
from jax.experimental import pallas as pl
from jax.experimental.pallas import tpu as pltpu
from jax.experimental.pallas import tpu_sc as plsc

_N = 10000
_E = 320000
_C = 128
_R = 8
_NB = 4
_RN = _R * _N

_NBLK = 10
_BLK = _N // _NBLK

_NW = 32
_CH = 80
_NCH = 126
_EPW = _NCH * _CH
_EPAD = _NW * _EPW
_NTRASH = _N
_NACC = _N + 256
_RPS = 624
_RREM = _N - 16 * _RPS


def _tables_body(x_ref, basis_ref, att_ref, root_ref, bias_ref,
                 table_ref, seed_ref):
    basis = basis_ref[...].reshape(_NB, _C * _C)
    w = jnp.dot(att_ref[...], basis,
                preferred_element_type=jnp.float32).reshape(_R, _C, _C)
    x = x_ref[...]
    for r in range(_R):
        table_ref[r] = jnp.dot(x, w[r], preferred_element_type=jnp.float32)
    seed_ref[...] = 0.5 * (
        jnp.dot(x, root_ref[...], preferred_element_type=jnp.float32)
        + bias_ref[...])


def _tables(x, basis, att, root, bias):
    return pl.pallas_call(
        _tables_body,
        grid=(_NBLK,),
        in_specs=[
            pl.BlockSpec((_BLK, _C), lambda i: (i, 0)),
            pl.BlockSpec((_NB, _C, _C), lambda i: (0, 0, 0)),
            pl.BlockSpec((_R, _NB), lambda i: (0, 0)),
            pl.BlockSpec((_C, _C), lambda i: (0, 0)),
            pl.BlockSpec((_C,), lambda i: (0,)),
        ],
        out_specs=[
            pl.BlockSpec((_R, _BLK, _C), lambda i: (0, i, 0)),
            pl.BlockSpec((_BLK, _C), lambda i: (i, 0)),
        ],
        out_shape=[
            jax.ShapeDtypeStruct((_R, _N, _C), jnp.float32),
            jax.ShapeDtypeStruct((_N, _C), jnp.float32),
        ],
    )(x, basis, att, root, bias)


def _tables2_body(p_ref, basis_ref, att_ref, root_ref, bias_ref,
                  table_ref, seed_ref):
    basis = basis_ref[...].reshape(_NB, _C * _C)
    w = jnp.dot(att_ref[...], basis,
                preferred_element_type=jnp.float32).reshape(_R, _C, _C)
    x = jnp.maximum(p_ref[0] + p_ref[1], 0.0)
    for r in range(_R):
        table_ref[r] = jnp.dot(x, w[r], preferred_element_type=jnp.float32)
    seed_ref[...] = 0.5 * (
        jnp.dot(x, root_ref[...], preferred_element_type=jnp.float32)
        + bias_ref[...])


def _tables2(p, basis, att, root, bias):
    return pl.pallas_call(
        _tables2_body,
        grid=(_NBLK,),
        in_specs=[
            pl.BlockSpec((2, _BLK, _C), lambda i: (0, i, 0)),
            pl.BlockSpec((_NB, _C, _C), lambda i: (0, 0, 0)),
            pl.BlockSpec((_R, _NB), lambda i: (0, 0)),
            pl.BlockSpec((_C, _C), lambda i: (0, 0)),
            pl.BlockSpec((_C,), lambda i: (0,)),
        ],
        out_specs=[
            pl.BlockSpec((_R, _BLK, _C), lambda i: (0, i, 0)),
            pl.BlockSpec((_BLK, _C), lambda i: (i, 0)),
        ],
        out_shape=[
            jax.ShapeDtypeStruct((_R, _N, _C), jnp.float32),
            jax.ShapeDtypeStruct((_N, _C), jnp.float32),
        ],
    )(p, basis, att, root, bias)


def _combine_body(p_ref, o_ref):
    o_ref[...] = p_ref[0] + p_ref[1]


def _combine(p):
    return pl.pallas_call(
        _combine_body,
        grid=(_NBLK,),
        in_specs=[pl.BlockSpec((2, _BLK, _C), lambda i: (0, i, 0))],
        out_specs=pl.BlockSpec((_BLK, _C), lambda i: (i, 0)),
        out_shape=jax.ShapeDtypeStruct((_N, _C), jnp.float32),
    )(p)


@functools.partial(
    pl.kernel,
    out_type=jax.ShapeDtypeStruct((2, _N, _C), jnp.float32),
    mesh=plsc.VectorSubcoreMesh(core_axis_name="c", subcore_axis_name="s"),
    scratch_types=[
        pltpu.VMEM((_EPW,), jnp.int32),
        pltpu.VMEM((_NCH, _CH), jnp.int32),
        pltpu.VMEM((_CH, _C), jnp.float32),
        pltpu.VMEM((_CH, _C), jnp.float32),
        pltpu.VMEM_SHARED((_NACC, _C), jnp.float32),
        pltpu.SemaphoreType.DMA,
        pltpu.SemaphoreType.DMA,
    ],
)
def _edge_scatter(table_hbm, gidx_hbm, dst_hbm, seed_hbm, out_hbm,
                  gidx_v, dst_v, rows_a, rows_b, acc_sh, sem_a, sem_b):
    cid = lax.axis_index("c")
    sid = lax.axis_index("s")
    wid = sid * 2 + cid

    pltpu.sync_copy(seed_hbm.at[pl.ds(sid * _RPS, _RPS)],
                    acc_sh.at[pl.ds(sid * _RPS, _RPS)])

    @pl.when(sid == 0)
    def _():
        pltpu.sync_copy(seed_hbm.at[pl.ds(16 * _RPS, _RREM)],
                        acc_sh.at[pl.ds(16 * _RPS, _RREM)])

    base = wid * _EPW
    pltpu.sync_copy(gidx_hbm.at[pl.ds(base, _EPW)], gidx_v)
    pltpu.sync_copy(dst_hbm.at[wid], dst_v)

    plsc.subcore_barrier()

    def gather_start(j, buf, sem):
        pltpu.async_copy(table_hbm.at[gidx_v.at[pl.ds(j * _CH, _CH)]],
                         buf, sem)

    def finish(j, buf, sem):
        pltpu.make_async_copy(table_hbm.at[gidx_v.at[pl.ds(j * _CH, _CH)]],
                              buf, sem).wait()
        pltpu.sync_copy(buf, acc_sh.at[dst_v.at[j]], add=True)

    gather_start(0, rows_a, sem_a)

    def body(i, carry):
        j = 2 * i
        gather_start(j + 1, rows_b, sem_b)
        finish(j, rows_a, sem_a)

        @pl.when(j + 2 < _NCH)
        def _():
            gather_start(j + 2, rows_a, sem_a)

        finish(j + 1, rows_b, sem_b)
        return carry

    lax.fori_loop(0, _NCH // 2, body, 0)

    plsc.subcore_barrier()

    pltpu.sync_copy(acc_sh.at[pl.ds(sid * _RPS, _RPS)],
                    out_hbm.at[cid, pl.ds(sid * _RPS, _RPS)])

    @pl.when(sid == 0)
    def _():
        pltpu.sync_copy(acc_sh.at[pl.ds(16 * _RPS, _RREM)],
                        out_hbm.at[cid, pl.ds(16 * _RPS, _RREM)])


def kernel(x, edge_index, edge_attr, basis0, att0, root0, bias0,
           basis1, att1, root1, bias1):
    src = edge_index[0].astype(jnp.int32)
    dst = edge_index[1].astype(jnp.int32)
    gidx = edge_attr.astype(jnp.int32) * _N + src
    ppw = _EPW - _E // _NW
    gidx = jnp.concatenate(
        [gidx.reshape(_NW, _E // _NW),
         jnp.zeros((_NW, ppw), jnp.int32)], axis=1).reshape(_EPAD)
    sid = jnp.arange(_NW, dtype=jnp.int32)[:, None] // 2
    trash = (_NTRASH + sid * 16
             + (jnp.arange(ppw, dtype=jnp.int32)[None, :] % 16))
    dst = jnp.concatenate(
        [dst.reshape(_NW, _E // _NW), trash], axis=1)
    dst = dst.reshape(_NW, _NCH, _CH)

    table0, seed0 = _tables(x, basis0, att0, root0, bias0)
    p0 = _edge_scatter(table0.reshape(_RN, _C), gidx, dst, seed0)
    table1, seed1 = _tables2(p0, basis1, att1, root1, bias1)
    p1 = _edge_scatter(table1.reshape(_RN, _C), gidx, dst, seed1)
    return _combine(p1)


# speedup vs baseline: 1.5722x; 1.5722x over previous
import functools

import jax
import jax.numpy as jnp
from jax import lax
from jax.experimental import pallas as pl
from jax.experimental.pallas import tpu as pltpu
from jax.experimental.pallas import tpu_sc as plsc

_N = 10000
_E = 320000
_C = 128
_R = 8
_NB = 4
_RN = _R * _N

_NBLK = 10
_BLK = _N // _NBLK

_NW = 32
_CH = 80
_NCH = 125
_EPW = _NCH * _CH
_NACC = _N
_RPS = 624
_RREM = _N - 16 * _RPS


def _tables_body(x_ref, basis_ref, att_ref, root_ref, bias_ref,
                 table_ref, seed_ref):
    basis = basis_ref[...].reshape(_NB, _C * _C)
    w = jnp.dot(att_ref[...], basis,
                preferred_element_type=jnp.float32).reshape(_R, _C, _C)
    x = x_ref[...]
    for r in range(_R):
        table_ref[r] = jnp.dot(x, w[r], preferred_element_type=jnp.float32)
    seed_ref[...] = 0.5 * (
        jnp.dot(x, root_ref[...], preferred_element_type=jnp.float32)
        + bias_ref[...])


def _tables(x, basis, att, root, bias):
    return pl.pallas_call(
        _tables_body,
        grid=(_NBLK,),
        in_specs=[
            pl.BlockSpec((_BLK, _C), lambda i: (i, 0)),
            pl.BlockSpec((_NB, _C, _C), lambda i: (0, 0, 0)),
            pl.BlockSpec((_R, _NB), lambda i: (0, 0)),
            pl.BlockSpec((_C, _C), lambda i: (0, 0)),
            pl.BlockSpec((_C,), lambda i: (0,)),
        ],
        out_specs=[
            pl.BlockSpec((_R, _BLK, _C), lambda i: (0, i, 0)),
            pl.BlockSpec((_BLK, _C), lambda i: (i, 0)),
        ],
        out_shape=[
            jax.ShapeDtypeStruct((_R, _N, _C), jnp.float32),
            jax.ShapeDtypeStruct((_N, _C), jnp.float32),
        ],
    )(x, basis, att, root, bias)


def _tables2_body(p_ref, basis_ref, att_ref, root_ref, bias_ref,
                  table_ref, seed_ref):
    basis = basis_ref[...].reshape(_NB, _C * _C)
    w = jnp.dot(att_ref[...], basis,
                preferred_element_type=jnp.float32).reshape(_R, _C, _C)
    x = jnp.maximum(p_ref[0] + p_ref[1], 0.0)
    for r in range(_R):
        table_ref[r] = jnp.dot(x, w[r], preferred_element_type=jnp.float32)
    seed_ref[...] = 0.5 * (
        jnp.dot(x, root_ref[...], preferred_element_type=jnp.float32)
        + bias_ref[...])


def _tables2(p, basis, att, root, bias):
    return pl.pallas_call(
        _tables2_body,
        grid=(_NBLK,),
        in_specs=[
            pl.BlockSpec((2, _BLK, _C), lambda i: (0, i, 0)),
            pl.BlockSpec((_NB, _C, _C), lambda i: (0, 0, 0)),
            pl.BlockSpec((_R, _NB), lambda i: (0, 0)),
            pl.BlockSpec((_C, _C), lambda i: (0, 0)),
            pl.BlockSpec((_C,), lambda i: (0,)),
        ],
        out_specs=[
            pl.BlockSpec((_R, _BLK, _C), lambda i: (0, i, 0)),
            pl.BlockSpec((_BLK, _C), lambda i: (i, 0)),
        ],
        out_shape=[
            jax.ShapeDtypeStruct((_R, _N, _C), jnp.float32),
            jax.ShapeDtypeStruct((_N, _C), jnp.float32),
        ],
    )(p, basis, att, root, bias)


def _combine_body(p_ref, o_ref):
    o_ref[...] = p_ref[0] + p_ref[1]


def _combine(p):
    return pl.pallas_call(
        _combine_body,
        grid=(_NBLK,),
        in_specs=[pl.BlockSpec((2, _BLK, _C), lambda i: (0, i, 0))],
        out_specs=pl.BlockSpec((_BLK, _C), lambda i: (i, 0)),
        out_shape=jax.ShapeDtypeStruct((_N, _C), jnp.float32),
    )(p)


@functools.partial(
    pl.kernel,
    out_type=jax.ShapeDtypeStruct((2, _N, _C), jnp.float32),
    mesh=plsc.VectorSubcoreMesh(core_axis_name="c", subcore_axis_name="s"),
    scratch_types=[
        pltpu.VMEM((_EPW,), jnp.int32),
        pltpu.VMEM((_NCH, _CH), jnp.int32),
        pltpu.VMEM((_CH, _C), jnp.float32),
        pltpu.VMEM((_CH, _C), jnp.float32),
        pltpu.VMEM_SHARED((_NACC, _C), jnp.float32),
        pltpu.SemaphoreType.DMA,
        pltpu.SemaphoreType.DMA,
    ],
)
def _edge_scatter(table_hbm, gidx_hbm, dst_hbm, seed_hbm, out_hbm,
                  gidx_v, dst_v, rows_a, rows_b, acc_sh, sem_a, sem_b):
    cid = lax.axis_index("c")
    sid = lax.axis_index("s")
    wid = sid * 2 + cid

    pltpu.sync_copy(seed_hbm.at[pl.ds(sid * _RPS, _RPS)],
                    acc_sh.at[pl.ds(sid * _RPS, _RPS)])

    @pl.when(sid == 0)
    def _():
        pltpu.sync_copy(seed_hbm.at[pl.ds(16 * _RPS, _RREM)],
                        acc_sh.at[pl.ds(16 * _RPS, _RREM)])

    base = wid * _EPW
    pltpu.sync_copy(gidx_hbm.at[pl.ds(base, _EPW)], gidx_v)
    pltpu.sync_copy(dst_hbm.at[wid], dst_v)

    plsc.subcore_barrier()

    def gather_start(j, buf, sem):
        pltpu.async_copy(table_hbm.at[gidx_v.at[pl.ds(j * _CH, _CH)]],
                         buf, sem)

    def finish(j, buf, sem):
        pltpu.make_async_copy(table_hbm.at[gidx_v.at[pl.ds(j * _CH, _CH)]],
                              buf, sem).wait()
        pltpu.sync_copy(buf, acc_sh.at[dst_v.at[j]], add=True)

    gather_start(0, rows_a, sem_a)

    def body(i, carry):
        j = 2 * i
        gather_start(j + 1, rows_b, sem_b)
        finish(j, rows_a, sem_a)

        @pl.when(j + 2 < _NCH)
        def _():
            gather_start(j + 2, rows_a, sem_a)

        finish(j + 1, rows_b, sem_b)
        return carry

    lax.fori_loop(0, _NCH // 2, body, 0)
    finish(_NCH - 1, rows_a, sem_a)

    plsc.subcore_barrier()

    pltpu.sync_copy(acc_sh.at[pl.ds(sid * _RPS, _RPS)],
                    out_hbm.at[cid, pl.ds(sid * _RPS, _RPS)])

    @pl.when(sid == 0)
    def _():
        pltpu.sync_copy(acc_sh.at[pl.ds(16 * _RPS, _RREM)],
                        out_hbm.at[cid, pl.ds(16 * _RPS, _RREM)])


def kernel(x, edge_index, edge_attr, basis0, att0, root0, bias0,
           basis1, att1, root1, bias1):
    src = edge_index[0].astype(jnp.int32)
    dst = edge_index[1].astype(jnp.int32)
    gidx = edge_attr.astype(jnp.int32) * _N + src
    dst = dst.reshape(_NW, _NCH, _CH)

    table0, seed0 = _tables(x, basis0, att0, root0, bias0)
    p0 = _edge_scatter(table0.reshape(_RN, _C), gidx, dst, seed0)
    table1, seed1 = _tables2(p0, basis1, att1, root1, bias1)
    p1 = _edge_scatter(table1.reshape(_RN, _C), gidx, dst, seed1)
    return _combine(p1)
